# baseline (speedup 1.0000x reference)
import functools

import jax
import jax.numpy as jnp
from jax.experimental import pallas as pl
from jax.experimental.pallas import tpu as pltpu

HIDDEN = 768
NUM_EXPERTS = 8
TOP_K = 2

BLK = 1024


def _router_body(x_ref, wt_ref, logits_ref, idx_ref, w_ref):
    xb = x_ref[...]
    wt = wt_ref[...]
    logits = jnp.dot(xb, wt, preferred_element_type=jnp.float32)
    logits_ref[...] = logits

    iota = jax.lax.broadcasted_iota(jnp.int32, logits.shape, 1)
    m1 = jnp.max(logits, axis=1, keepdims=True)
    i1 = jnp.min(jnp.where(logits == m1, iota, NUM_EXPERTS), axis=1,
                 keepdims=True)
    masked = jnp.where(iota == i1, -1e30, logits)
    m2 = jnp.max(masked, axis=1, keepdims=True)
    i2 = jnp.min(jnp.where(masked == m2, iota, NUM_EXPERTS), axis=1,
                 keepdims=True)
    e = jnp.exp(m2 - m1)
    w1 = 1.0 / (1.0 + e)
    w2 = 1.0 - w1
    idx_ref[...] = jnp.concatenate([i1, i2], axis=1)
    w_ref[...] = jnp.concatenate([w1, w2], axis=1)


@jax.jit
def kernel(x, W):
    b, s, h = x.shape
    n = b * s
    x_flat = x.reshape(n, h)
    wt = W.T
    grid = (n // BLK,)
    logits, idx, w = pl.pallas_call(
        _router_body,
        grid=grid,
        in_specs=[
            pl.BlockSpec((BLK, h), lambda i: (i, 0)),
            pl.BlockSpec((h, NUM_EXPERTS), lambda i: (0, 0)),
        ],
        out_specs=[
            pl.BlockSpec((BLK, NUM_EXPERTS), lambda i: (i, 0)),
            pl.BlockSpec((BLK, TOP_K), lambda i: (i, 0)),
            pl.BlockSpec((BLK, TOP_K), lambda i: (i, 0)),
        ],
        out_shape=[
            jax.ShapeDtypeStruct((n, NUM_EXPERTS), jnp.float32),
            jax.ShapeDtypeStruct((n, TOP_K), jnp.int32),
            jax.ShapeDtypeStruct((n, TOP_K), jnp.float32),
        ],
    )(x_flat, wt)
    return (logits, idx, w)


# speedup vs baseline: 1.6309x; 1.6309x over previous
import functools

import jax
import jax.numpy as jnp
from jax.experimental import pallas as pl
from jax.experimental.pallas import tpu as pltpu

HIDDEN = 768
NUM_EXPERTS = 8
TOP_K = 2

BLK = 1024


def _router_body(x_ref, wt_ref, logits_ref, idx_ref, w_ref):
    xb = x_ref[...]
    wt = wt_ref[...]
    logits = jnp.dot(xb, wt, preferred_element_type=jnp.float32)
    logits_ref[...] = logits

    lt = logits.T
    iota = jax.lax.broadcasted_iota(jnp.int32, lt.shape, 0)
    m1 = jnp.max(lt, axis=0, keepdims=True)
    i1 = jnp.min(jnp.where(lt == m1, iota, NUM_EXPERTS), axis=0,
                 keepdims=True)
    masked = jnp.where(iota == i1, -1e30, lt)
    m2 = jnp.max(masked, axis=0, keepdims=True)
    i2 = jnp.min(jnp.where(masked == m2, iota, NUM_EXPERTS), axis=0,
                 keepdims=True)
    e = jnp.exp(m2 - m1)
    w1 = 1.0 / (1.0 + e)
    w2 = 1.0 - w1
    idx_ref[...] = jnp.concatenate([i1, i2], axis=0)
    w_ref[...] = jnp.concatenate([w1, w2], axis=0)


@jax.jit
def kernel(x, W):
    b, s, h = x.shape
    n = b * s
    x_flat = x.reshape(n, h)
    wt = W.T
    grid = (n // BLK,)
    logits, idx_t, w_t = pl.pallas_call(
        _router_body,
        grid=grid,
        in_specs=[
            pl.BlockSpec((BLK, h), lambda i: (i, 0)),
            pl.BlockSpec((h, NUM_EXPERTS), lambda i: (0, 0)),
        ],
        out_specs=[
            pl.BlockSpec((BLK, NUM_EXPERTS), lambda i: (i, 0)),
            pl.BlockSpec((TOP_K, BLK), lambda i: (0, i)),
            pl.BlockSpec((TOP_K, BLK), lambda i: (0, i)),
        ],
        out_shape=[
            jax.ShapeDtypeStruct((n, NUM_EXPERTS), jnp.float32),
            jax.ShapeDtypeStruct((TOP_K, n), jnp.int32),
            jax.ShapeDtypeStruct((TOP_K, n), jnp.float32),
        ],
    )(x_flat, wt)
    return (logits, idx_t.T, w_t.T)


# speedup vs baseline: 1.8955x; 1.1623x over previous
import functools

import jax
import jax.numpy as jnp
from jax.experimental import pallas as pl
from jax.experimental.pallas import tpu as pltpu

HIDDEN = 768
NUM_EXPERTS = 8
TOP_K = 2

BLK = 2048


def _router_body(x_ref, wt_ref, logits_ref, idx_ref, w_ref):
    xb = x_ref[...]
    wt = wt_ref[...]
    logits = jnp.dot(xb, wt, preferred_element_type=jnp.float32)
    logits_ref[...] = logits

    lt = logits.T
    iota = jax.lax.broadcasted_iota(jnp.int32, lt.shape, 0)
    m1 = jnp.max(lt, axis=0, keepdims=True)
    i1 = jnp.min(jnp.where(lt == m1, iota, NUM_EXPERTS), axis=0,
                 keepdims=True)
    masked = jnp.where(iota == i1, -1e30, lt)
    m2 = jnp.max(masked, axis=0, keepdims=True)
    i2 = jnp.min(jnp.where(masked == m2, iota, NUM_EXPERTS), axis=0,
                 keepdims=True)
    e = jnp.exp(m2 - m1)
    w1 = 1.0 / (1.0 + e)
    w2 = 1.0 - w1
    idx_ref[...] = jnp.concatenate([i1, i2], axis=0)
    w_ref[...] = jnp.concatenate([w1, w2], axis=0)


@jax.jit
def kernel(x, W):
    b, s, h = x.shape
    n = b * s
    x_flat = x.reshape(n, h)
    wt = W.T
    grid = (n // BLK,)
    logits, idx_t, w_t = pl.pallas_call(
        _router_body,
        grid=grid,
        in_specs=[
            pl.BlockSpec((BLK, h), lambda i: (i, 0)),
            pl.BlockSpec((h, NUM_EXPERTS), lambda i: (0, 0)),
        ],
        out_specs=[
            pl.BlockSpec((BLK, NUM_EXPERTS), lambda i: (i, 0)),
            pl.BlockSpec((TOP_K, BLK), lambda i: (0, i)),
            pl.BlockSpec((TOP_K, BLK), lambda i: (0, i)),
        ],
        out_shape=[
            jax.ShapeDtypeStruct((n, NUM_EXPERTS), jnp.float32),
            jax.ShapeDtypeStruct((TOP_K, n), jnp.int32),
            jax.ShapeDtypeStruct((TOP_K, n), jnp.float32),
        ],
    )(x_flat, wt)
    return (logits, idx_t.T, w_t.T)


# speedup vs baseline: 1.9512x; 1.0294x over previous
import functools

import jax
import jax.numpy as jnp
from jax.experimental import pallas as pl
from jax.experimental.pallas import tpu as pltpu

HIDDEN = 768
NUM_EXPERTS = 8
TOP_K = 2

BLK = 4096


def _router_body(x_ref, wt_ref, logits_ref, idx_ref, w_ref):
    xb = x_ref[...]
    wt = wt_ref[...]
    logits = jnp.dot(xb, wt, preferred_element_type=jnp.float32)
    logits_ref[...] = logits

    lt = logits.T
    iota = jax.lax.broadcasted_iota(jnp.int32, lt.shape, 0)
    m1 = jnp.max(lt, axis=0, keepdims=True)
    i1 = jnp.min(jnp.where(lt == m1, iota, NUM_EXPERTS), axis=0,
                 keepdims=True)
    masked = jnp.where(iota == i1, -1e30, lt)
    m2 = jnp.max(masked, axis=0, keepdims=True)
    i2 = jnp.min(jnp.where(masked == m2, iota, NUM_EXPERTS), axis=0,
                 keepdims=True)
    e = jnp.exp(m2 - m1)
    w1 = 1.0 / (1.0 + e)
    w2 = 1.0 - w1
    idx_ref[...] = jnp.concatenate([i1, i2], axis=0)
    w_ref[...] = jnp.concatenate([w1, w2], axis=0)


@jax.jit
def kernel(x, W):
    b, s, h = x.shape
    n = b * s
    x_flat = x.reshape(n, h)
    wt = W.T
    grid = (n // BLK,)
    logits, idx_t, w_t = pl.pallas_call(
        _router_body,
        grid=grid,
        in_specs=[
            pl.BlockSpec((BLK, h), lambda i: (i, 0)),
            pl.BlockSpec((h, NUM_EXPERTS), lambda i: (0, 0)),
        ],
        out_specs=[
            pl.BlockSpec((BLK, NUM_EXPERTS), lambda i: (i, 0)),
            pl.BlockSpec((TOP_K, BLK), lambda i: (0, i)),
            pl.BlockSpec((TOP_K, BLK), lambda i: (0, i)),
        ],
        out_shape=[
            jax.ShapeDtypeStruct((n, NUM_EXPERTS), jnp.float32),
            jax.ShapeDtypeStruct((TOP_K, n), jnp.int32),
            jax.ShapeDtypeStruct((TOP_K, n), jnp.float32),
        ],
    )(x_flat, wt)
    return (logits, idx_t.T, w_t.T)
